# baseline (speedup 1.0000x reference)
import jax
import jax.numpy as jnp
from jax import lax
from jax.experimental import pallas as pl
from jax.experimental.pallas import tpu as pltpu
from jax.experimental.pallas import tpu_sc as plsc

ROWS_PER_BLOCK = 256
CHUNK = 256

NUM_SC_CORES = 2
NUM_SC_SUBCORES = 16
NUM_WORKERS = NUM_SC_CORES * NUM_SC_SUBCORES
LANES = 16
GROUP_ROWS = 8


def _tc_cumsum_kernel(x_ref, o_ref):
    width = x_ref.shape[1]
    nchunk = width // CHUNK
    ii = lax.broadcasted_iota(jnp.int32, (CHUNK, CHUNK), 0)
    jj = lax.broadcasted_iota(jnp.int32, (CHUNK, CHUNK), 1)
    tri = (ii <= jj).astype(jnp.bfloat16)
    carry = jnp.zeros((x_ref.shape[0], 1), jnp.float32)
    for c in range(nchunk):
        xc = x_ref[:, c * CHUNK:(c + 1) * CHUNK]
        hi = xc.astype(jnp.bfloat16)
        y = jnp.dot(hi, tri, preferred_element_type=jnp.float32)
        y = y + carry
        o_ref[:, c * CHUNK:(c + 1) * CHUNK] = y
        carry = y[:, CHUNK - 1:CHUNK]


def tc_cumsum(x):
    m, n = x.shape
    return pl.pallas_call(
        _tc_cumsum_kernel,
        grid=(m // ROWS_PER_BLOCK,),
        in_specs=[pl.BlockSpec((ROWS_PER_BLOCK, n), lambda i: (i, 0))],
        out_specs=pl.BlockSpec((ROWS_PER_BLOCK, n), lambda i: (i, 0)),
        out_shape=jax.ShapeDtypeStruct((m, n), x.dtype),
        compiler_params=pltpu.CompilerParams(
            dimension_semantics=("parallel",),
        ),
    )(x)


def _sc_body(x_hbm, o_hbm, buf):
    width = x_hbm.shape[1]
    nvec = width // LANES
    rows = x_hbm.shape[0]
    rows_per_worker = rows // NUM_WORKERS
    ngroups = rows_per_worker // GROUP_ROWS
    wid = lax.axis_index("s") * NUM_SC_CORES + lax.axis_index("c")

    def group_body(g, _):
        row0 = wid * rows_per_worker + g * GROUP_ROWS
        pltpu.sync_copy(x_hbm.at[pl.ds(row0, GROUP_ROWS), :], buf)
        for r in range(GROUP_ROWS):
            def vec_body(i, c):
                v = buf[r, pl.ds(i * LANES, LANES)]
                buf[r, pl.ds(i * LANES, LANES)] = plsc.cumsum(v) + c
                return c + jnp.sum(v)
            lax.fori_loop(0, nvec, vec_body, jnp.float32(0.0), unroll=8)
        pltpu.sync_copy(buf, o_hbm.at[pl.ds(row0, GROUP_ROWS), :])
        return 0

    lax.fori_loop(0, ngroups, group_body, 0)


def sc_cumsum(x):
    m, n = x.shape
    fn = pl.kernel(
        _sc_body,
        out_type=jax.ShapeDtypeStruct((m, n), x.dtype),
        mesh=plsc.VectorSubcoreMesh(core_axis_name="c", subcore_axis_name="s"),
        scratch_types=[pltpu.VMEM((GROUP_ROWS, n), jnp.float32)],
        compiler_params=pltpu.CompilerParams(needs_layout_passes=False),
    )
    return fn(x)


def kernel(x):
    return sc_cumsum(x)


# speedup vs baseline: 1.0570x; 1.0570x over previous
import jax
import jax.numpy as jnp
from jax import lax
from jax.experimental import pallas as pl
from jax.experimental.pallas import tpu as pltpu
from jax.experimental.pallas import tpu_sc as plsc

ROWS_PER_BLOCK = 256
CHUNK = 256

NUM_SC_CORES = 2
NUM_SC_SUBCORES = 16
NUM_WORKERS = NUM_SC_CORES * NUM_SC_SUBCORES
LANES = 16
GROUP_ROWS = 8


def _tc_cumsum_kernel(x_ref, o_ref):
    width = x_ref.shape[1]
    nchunk = width // CHUNK
    ii = lax.broadcasted_iota(jnp.int32, (CHUNK, CHUNK), 0)
    jj = lax.broadcasted_iota(jnp.int32, (CHUNK, CHUNK), 1)
    tri = (ii <= jj).astype(jnp.bfloat16)
    carry = jnp.zeros((x_ref.shape[0], 1), jnp.float32)
    for c in range(nchunk):
        xc = x_ref[:, c * CHUNK:(c + 1) * CHUNK]
        hi = xc.astype(jnp.bfloat16)
        y = jnp.dot(hi, tri, preferred_element_type=jnp.float32)
        y = y + carry
        o_ref[:, c * CHUNK:(c + 1) * CHUNK] = y
        carry = y[:, CHUNK - 1:CHUNK]


def tc_cumsum(x):
    m, n = x.shape
    return pl.pallas_call(
        _tc_cumsum_kernel,
        grid=(m // ROWS_PER_BLOCK,),
        in_specs=[pl.BlockSpec((ROWS_PER_BLOCK, n), lambda i: (i, 0))],
        out_specs=pl.BlockSpec((ROWS_PER_BLOCK, n), lambda i: (i, 0)),
        out_shape=jax.ShapeDtypeStruct((m, n), x.dtype),
        compiler_params=pltpu.CompilerParams(
            dimension_semantics=("parallel",),
        ),
    )(x)


def _sc_body(x_hbm, o_hbm, buf):
    width = x_hbm.shape[1]
    nvec = width // LANES
    rows = x_hbm.shape[0]
    rows_per_worker = rows // NUM_WORKERS
    ngroups = rows_per_worker // GROUP_ROWS
    wid = lax.axis_index("s") * NUM_SC_CORES + lax.axis_index("c")

    def group_body(g, _):
        row0 = wid * rows_per_worker + g * GROUP_ROWS
        pltpu.sync_copy(x_hbm.at[pl.ds(row0, GROUP_ROWS), :], buf)
        for r in range(GROUP_ROWS):
            def vec_body(i, c):
                v = buf[r, pl.ds(i * LANES, LANES)]
                buf[r, pl.ds(i * LANES, LANES)] = plsc.cumsum(v) + c
                return c + jnp.sum(v)
            lax.fori_loop(0, nvec, vec_body, jnp.float32(0.0), unroll=8)
        pltpu.sync_copy(buf, o_hbm.at[pl.ds(row0, GROUP_ROWS), :])
        return 0

    lax.fori_loop(0, ngroups, group_body, 0)


def sc_cumsum(x):
    m, n = x.shape
    fn = pl.kernel(
        _sc_body,
        out_type=jax.ShapeDtypeStruct((m, n), x.dtype),
        mesh=plsc.VectorSubcoreMesh(core_axis_name="c", subcore_axis_name="s"),
        scratch_types=[pltpu.VMEM((GROUP_ROWS, n), jnp.float32)],
        compiler_params=pltpu.CompilerParams(needs_layout_passes=False),
    )
    return fn(x)


SC_ROWS = 768


def kernel(x):
    out_tc = tc_cumsum(x[:-SC_ROWS])
    out_sc = sc_cumsum(x[-SC_ROWS:])
    return jnp.concatenate([out_tc, out_sc], axis=0)


# speedup vs baseline: 3.3345x; 3.1548x over previous
import jax
import jax.numpy as jnp
from jax import lax
from jax.experimental import pallas as pl
from jax.experimental.pallas import tpu as pltpu

BR = 128
NBUF = 4
CHUNK = 256


def _compute(ibuf, obuf, islot, oslot, n):
    nchunk = n // CHUNK
    ii = lax.broadcasted_iota(jnp.int32, (CHUNK, CHUNK), 0)
    jj = lax.broadcasted_iota(jnp.int32, (CHUNK, CHUNK), 1)
    tri = (ii <= jj).astype(jnp.bfloat16)
    carry = jnp.zeros((BR, 1), jnp.float32)
    for c in range(nchunk):
        xc = ibuf[islot, :, c * CHUNK:(c + 1) * CHUNK]
        y = jnp.dot(xc.astype(jnp.bfloat16), tri,
                    preferred_element_type=jnp.float32)
        y = y + carry
        obuf[oslot, :, c * CHUNK:(c + 1) * CHUNK] = y
        carry = y[:, CHUNK - 1:CHUNK]


def _cumsum_body(x_hbm, o_hbm, ibuf, obuf, isem, osem):
    m, n = x_hbm.shape
    nstep = m // BR

    def in_copy(step):
        slot = step % NBUF
        return pltpu.make_async_copy(
            x_hbm.at[pl.ds(step * BR, BR), :], ibuf.at[slot], isem.at[slot])

    def out_copy(step):
        slot = step % NBUF
        return pltpu.make_async_copy(
            obuf.at[slot], o_hbm.at[pl.ds(step * BR, BR), :], osem.at[slot])

    for s in range(NBUF):
        in_copy(s).start()
    for step in range(nstep):
        in_copy(step).wait()
        if step >= NBUF:
            out_copy(step - NBUF).wait()
        _compute(ibuf, obuf, step % NBUF, step % NBUF, n)
        out_copy(step).start()
        if step + NBUF < nstep:
            in_copy(step + NBUF).start()
    for step in range(nstep - NBUF, nstep):
        out_copy(step).wait()


def kernel(x):
    m, n = x.shape
    return pl.pallas_call(
        _cumsum_body,
        in_specs=[pl.BlockSpec(memory_space=pl.ANY)],
        out_specs=pl.BlockSpec(memory_space=pl.ANY),
        out_shape=jax.ShapeDtypeStruct((m, n), x.dtype),
        scratch_shapes=[
            pltpu.VMEM((NBUF, BR, n), jnp.float32),
            pltpu.VMEM((NBUF, BR, n), jnp.float32),
            pltpu.SemaphoreType.DMA((NBUF,)),
            pltpu.SemaphoreType.DMA((NBUF,)),
        ],
    )(x)
